# speedup vs baseline: 2.4130x; 2.4130x over previous
import functools
import math

import jax
import jax.numpy as jnp
from jax import lax
from jax.experimental import pallas as pl
from jax.experimental.pallas import tpu as pltpu
from jax.experimental.pallas import tpu_sc as plsc

VOCAB = 100000
D = 128
B = 4096
L = 50

NC = 2
NS = 16
LANES = 16
NW = NC * NS

N = B * L
PW = N // NW
R = 128
CHUNKS = PW // R

SCALE = math.sqrt(D)

_mesh = plsc.VectorSubcoreMesh(core_axis_name="c", subcore_axis_name="s")


@functools.partial(
    pl.kernel,
    out_type=jax.ShapeDtypeStruct((N, D), jnp.float32),
    mesh=_mesh,
    scratch_types=[
        pltpu.VMEM((CHUNKS, R), jnp.int32),
        pltpu.VMEM((R, D), jnp.float32),
        pltpu.SemaphoreType.DMA,
    ],
)
def _emb_kernel(x_hbm, table_hbm, out_hbm, idx_v, rows_v, sem):
    wid = lax.axis_index("s") * NC + lax.axis_index("c")
    pltpu.sync_copy(x_hbm.at[wid], idx_v)
    row_base = wid * PW

    def chunk_body(c, carry):
        pltpu.async_copy(table_hbm.at[idx_v.at[c]], rows_v, sem).wait()

        def scale_row(i, carry2):
            for j in range(D // LANES):
                sl = pl.ds(j * LANES, LANES)
                rows_v[i, sl] = rows_v[i, sl] * SCALE
            return carry2

        lax.fori_loop(0, R, scale_row, 0)
        pltpu.sync_copy(rows_v, out_hbm.at[pl.ds(row_base + c * R, R)])
        return carry

    lax.fori_loop(0, CHUNKS, chunk_body, 0)


def kernel(x, table):
    x_flat = x.reshape(NW, CHUNKS, R).astype(jnp.int32)
    out = _emb_kernel(x_flat, table)
    return out.reshape(B, L, D)


# speedup vs baseline: 2.9455x; 1.2207x over previous
import functools
import math

import jax
import jax.numpy as jnp
from jax import lax
from jax.experimental import pallas as pl
from jax.experimental.pallas import tpu as pltpu
from jax.experimental.pallas import tpu_sc as plsc

VOCAB = 100000
D = 128
B = 4096
L = 50

NC = 2
NS = 16
LANES = 16
NW = NC * NS

N = B * L
PW = N // NW
R = 128
CHUNKS = PW // R
NB = 5

SCALE = math.sqrt(D)

_mesh = plsc.VectorSubcoreMesh(core_axis_name="c", subcore_axis_name="s")

_scratch = (
    [pltpu.VMEM((CHUNKS, R), jnp.int32)]
    + [pltpu.VMEM((R, D), jnp.float32) for _ in range(NB)]
    + [pltpu.SemaphoreType.DMA for _ in range(2 * NB)]
)


@functools.partial(
    pl.kernel,
    out_type=jax.ShapeDtypeStruct((N, D), jnp.float32),
    mesh=_mesh,
    scratch_types=_scratch,
)
def _emb_kernel(x_hbm, table_hbm, out_hbm, idx_v, *scratch):
    bufs = scratch[:NB]
    gsems = scratch[NB:2 * NB]
    osems = scratch[2 * NB:]

    wid = lax.axis_index("s") * NC + lax.axis_index("c")
    pltpu.sync_copy(x_hbm.at[wid], idx_v)
    row_base = wid * PW

    def start_gather(c, s):
        pltpu.async_copy(table_hbm.at[idx_v.at[c]], bufs[s], gsems[s])

    def wait_gather(c, s):
        pltpu.make_async_copy(table_hbm.at[idx_v.at[c]], bufs[s], gsems[s]).wait()

    def out_slice(c):
        return out_hbm.at[pl.ds(row_base + c * R, R)]

    def start_out(c, s):
        pltpu.async_copy(bufs[s], out_slice(c), osems[s])

    def wait_out(c, s):
        pltpu.make_async_copy(bufs[s], out_slice(c), osems[s]).wait()

    def scale(s):
        buf = bufs[s]

        @plsc.parallel_loop(0, R, unroll=4)
        def _(i):
            for j in range(D // LANES):
                sl = pl.ds(j * LANES, LANES)
                buf[i, sl] = buf[i, sl] * SCALE

    for c in range(NB - 1):
        start_gather(c, c)
    wait_gather(0, 0)
    scale(0)
    start_out(0, 0)
    start_gather(NB - 1, NB - 1)

    def outer(g, carry):
        for b in range(NB):
            c = 1 + g * NB + b
            s = (1 + b) % NB
            wait_gather(c, s)
            scale(s)
            start_out(c, s)
            s2 = (s + NB - 1) % NB
            wait_out(c - 1, s2)
            start_gather(c + NB - 1, s2)
        return carry

    lax.fori_loop(0, (CHUNKS - NB) // NB, outer, 0)

    for c in range(CHUNKS - NB + 1, CHUNKS):
        s = c % NB
        wait_gather(c, s)
        scale(s)
        start_out(c, s)
    for c in range(CHUNKS - NB, CHUNKS):
        wait_out(c, c % NB)


def kernel(x, table):
    x_flat = x.reshape(NW, CHUNKS, R).astype(jnp.int32)
    out = _emb_kernel(x_flat, table)
    return out.reshape(B, L, D)


# speedup vs baseline: 5.2329x; 1.7766x over previous
import functools
import math

import jax
import jax.numpy as jnp
from jax import lax
from jax.experimental import pallas as pl
from jax.experimental.pallas import tpu as pltpu
from jax.experimental.pallas import tpu_sc as plsc

VOCAB = 100000
D = 128
B = 4096
L = 50

NC = 2
NS = 16
LANES = 16
NW = NC * NS

BW = B // NW
BPC = 2
RC = BPC * L
CHUNKS = BW // BPC
NB = 4

SCALE = math.sqrt(D)

_mesh = plsc.VectorSubcoreMesh(core_axis_name="c", subcore_axis_name="s")

_scratch = (
    [pltpu.VMEM((CHUNKS, RC), jnp.int32)]
    + [pltpu.VMEM((RC, D), jnp.float32) for _ in range(NB)]
    + [pltpu.SemaphoreType.DMA for _ in range(2 * NB)]
)


@functools.partial(
    pl.kernel,
    out_type=jax.ShapeDtypeStruct((B, L, D), jnp.float32),
    mesh=_mesh,
    scratch_types=_scratch,
    compiler_params=pltpu.CompilerParams(use_tc_tiling_on_sc=True),
)
def _emb_kernel(x_hbm, table_hbm, out_hbm, idx_v, *scratch):
    bufs = scratch[:NB]
    gsems = scratch[NB:2 * NB]
    osems = scratch[2 * NB:]

    wid = lax.axis_index("s") * NC + lax.axis_index("c")
    pltpu.sync_copy(x_hbm.at[wid], idx_v)
    batch_base = wid * BW

    def start_gather(c, s):
        pltpu.async_copy(table_hbm.at[idx_v.at[c]], bufs[s], gsems[s])

    def wait_gather(c, s):
        pltpu.make_async_copy(table_hbm.at[idx_v.at[c]], bufs[s], gsems[s]).wait()

    def start_out(c, s):
        for k in range(BPC):
            pltpu.async_copy(
                bufs[s].at[pl.ds(k * L, L)],
                out_hbm.at[batch_base + c * BPC + k],
                osems[s],
            )

    def wait_out(c, s):
        for k in range(BPC):
            pltpu.make_async_copy(
                bufs[s].at[pl.ds(k * L, L)],
                out_hbm.at[batch_base + c * BPC + k],
                osems[s],
            ).wait()

    def scale(s):
        buf = bufs[s]

        @plsc.parallel_loop(0, RC, unroll=4)
        def _(i):
            for j in range(D // LANES):
                sl = pl.ds(j * LANES, LANES)
                buf[i, sl] = buf[i, sl] * SCALE

    for c in range(NB - 1):
        start_gather(c, c)
    wait_gather(0, 0)
    scale(0)
    start_out(0, 0)
    start_gather(NB - 1, NB - 1)

    def outer(g, carry):
        for b in range(NB):
            c = 1 + g * NB + b
            s = (1 + b) % NB
            wait_gather(c, s)
            scale(s)
            start_out(c, s)
            s2 = (s + NB - 1) % NB
            wait_out(c - 1, s2)
            start_gather(c + NB - 1, s2)
        return carry

    lax.fori_loop(0, (CHUNKS - NB) // NB, outer, 0)

    for c in range(CHUNKS - NB + 1, CHUNKS):
        s = c % NB
        wait_gather(c, s)
        scale(s)
        start_out(c, s)
    for c in range(CHUNKS - NB, CHUNKS):
        wait_out(c, c % NB)


def kernel(x, table):
    x_flat = x.reshape(NW, CHUNKS, RC).astype(jnp.int32)
    return _emb_kernel(x_flat, table)


# speedup vs baseline: 8.7711x; 1.6761x over previous
import functools
import math

import jax
import jax.numpy as jnp
import numpy as np
from jax import lax
from jax.experimental import pallas as pl
from jax.experimental.pallas import tpu as pltpu
from jax.experimental.pallas import tpu_sc as plsc

VOCAB = 100000
D = 128
B = 4096
L = 50

NC = 2
NS = 16
LANES = 16
NW = NC * NS

N = B * L
PW = N // NW
R = 128
CHUNKS = PW // R
NB = 5

SCALE = math.sqrt(D)

_n = np.arange(N, dtype=np.int64)
_DST = ((_n % L) * B + _n // L).astype(np.int32).reshape(NW, CHUNKS, R)

_mesh = plsc.VectorSubcoreMesh(core_axis_name="c", subcore_axis_name="s")

_scratch = (
    [pltpu.VMEM((CHUNKS, R), jnp.int32) for _ in range(2)]
    + [pltpu.VMEM((R, D), jnp.float32) for _ in range(NB)]
    + [pltpu.SemaphoreType.DMA for _ in range(2 * NB)]
)


@functools.partial(
    pl.kernel,
    out_type=jax.ShapeDtypeStruct((N, D), jnp.float32),
    mesh=_mesh,
    scratch_types=_scratch,
    compiler_params=pltpu.CompilerParams(use_tc_tiling_on_sc=True),
)
def _emb_kernel(x_hbm, dst_hbm, table_hbm, out_hbm, idx_v, dst_v, *scratch):
    bufs = scratch[:NB]
    gsems = scratch[NB:2 * NB]
    osems = scratch[2 * NB:]

    wid = lax.axis_index("s") * NC + lax.axis_index("c")
    pltpu.sync_copy(x_hbm.at[wid], idx_v)
    pltpu.sync_copy(dst_hbm.at[wid], dst_v)

    def start_gather(c, s):
        pltpu.async_copy(table_hbm.at[idx_v.at[c]], bufs[s], gsems[s])

    def wait_gather(c, s):
        pltpu.make_async_copy(table_hbm.at[idx_v.at[c]], bufs[s], gsems[s]).wait()

    def start_out(c, s):
        pltpu.async_copy(bufs[s], out_hbm.at[dst_v.at[c]], osems[s])

    def wait_out(c, s):
        pltpu.make_async_copy(bufs[s], out_hbm.at[dst_v.at[c]], osems[s]).wait()

    def scale(s):
        buf = bufs[s]

        @plsc.parallel_loop(0, R, unroll=4)
        def _(i):
            for j in range(D // LANES):
                sl = pl.ds(j * LANES, LANES)
                buf[i, sl] = buf[i, sl] * SCALE

    for c in range(NB - 1):
        start_gather(c, c)
    wait_gather(0, 0)
    scale(0)
    start_out(0, 0)
    start_gather(NB - 1, NB - 1)

    def outer(g, carry):
        for b in range(NB):
            c = 1 + g * NB + b
            s = (1 + b) % NB
            wait_gather(c, s)
            scale(s)
            start_out(c, s)
            s2 = (s + NB - 1) % NB
            wait_out(c - 1, s2)
            start_gather(c + NB - 1, s2)
        return carry

    lax.fori_loop(0, (CHUNKS - NB) // NB, outer, 0)

    for c in range(CHUNKS - NB + 1, CHUNKS):
        s = c % NB
        wait_gather(c, s)
        scale(s)
        start_out(c, s)
    for c in range(CHUNKS - NB, CHUNKS):
        wait_out(c, c % NB)


def kernel(x, table):
    x_flat = x.reshape(NW, CHUNKS, R).astype(jnp.int32)
    out = _emb_kernel(x_flat, jnp.asarray(_DST), table)
    return out.reshape(L, B, D).transpose(1, 0, 2)


# speedup vs baseline: 8.9149x; 1.0164x over previous
import functools
import math

import jax
import jax.numpy as jnp
import numpy as np
from jax import lax
from jax.experimental import pallas as pl
from jax.experimental.pallas import tpu as pltpu
from jax.experimental.pallas import tpu_sc as plsc

VOCAB = 100000
D = 128
B = 4096
L = 50

NC = 2
NS = 16
LANES = 16
NW = NC * NS

N = B * L
PW = N // NW
R = 128
CHUNKS = PW // R
NB = 5

SCALE = math.sqrt(D)

_n = np.arange(N, dtype=np.int64)
_DST = ((_n % L) * B + _n // L).astype(np.int32).reshape(NW, CHUNKS, R)

_mesh = plsc.VectorSubcoreMesh(core_axis_name="c", subcore_axis_name="s")

_scratch = (
    [pltpu.VMEM((CHUNKS, R), jnp.int32) for _ in range(2)]
    + [pltpu.VMEM((R, D), jnp.float32) for _ in range(NB)]
    + [pltpu.SemaphoreType.DMA for _ in range(2 * NB)]
)


@functools.partial(
    pl.kernel,
    out_type=jax.ShapeDtypeStruct((N, D), jnp.float32),
    mesh=_mesh,
    scratch_types=_scratch,
    compiler_params=pltpu.CompilerParams(use_tc_tiling_on_sc=True),
)
def _emb_kernel(x_hbm, dst_hbm, table_hbm, out_hbm, idx_v, dst_v, *scratch):
    bufs = scratch[:NB]
    gsems = scratch[NB:2 * NB]
    osems = scratch[2 * NB:]

    wid = lax.axis_index("s") * NC + lax.axis_index("c")
    pltpu.sync_copy(x_hbm.at[wid], idx_v)
    pltpu.sync_copy(dst_hbm.at[wid], dst_v)

    def start_gather(c, s):
        pltpu.async_copy(table_hbm.at[idx_v.at[c]], bufs[s], gsems[s])

    def wait_gather(c, s):
        pltpu.make_async_copy(table_hbm.at[idx_v.at[c]], bufs[s], gsems[s]).wait()

    def start_out(c, s):
        pltpu.async_copy(bufs[s], out_hbm.at[dst_v.at[c]], osems[s])

    def wait_out(c, s):
        pltpu.make_async_copy(bufs[s], out_hbm.at[dst_v.at[c]], osems[s]).wait()

    def scale(s):
        buf = bufs[s]

        @plsc.parallel_loop(0, 1, unroll=1)
        def _(i):
            for j in range(D // LANES):
                sl = pl.ds(j * LANES, LANES)
                buf[i, sl] = buf[i, sl] * SCALE

    for c in range(NB - 1):
        start_gather(c, c)
    wait_gather(0, 0)
    scale(0)
    start_out(0, 0)
    start_gather(NB - 1, NB - 1)

    def outer(g, carry):
        for b in range(NB):
            c = 1 + g * NB + b
            s = (1 + b) % NB
            wait_gather(c, s)
            scale(s)
            start_out(c, s)
            s2 = (s + NB - 1) % NB
            wait_out(c - 1, s2)
            start_gather(c + NB - 1, s2)
        return carry

    lax.fori_loop(0, (CHUNKS - NB) // NB, outer, 0)

    for c in range(CHUNKS - NB + 1, CHUNKS):
        s = c % NB
        wait_gather(c, s)
        scale(s)
        start_out(c, s)
    for c in range(CHUNKS - NB, CHUNKS):
        wait_out(c, c % NB)


def kernel(x, table):
    x_flat = x.reshape(NW, CHUNKS, R).astype(jnp.int32)
    out = _emb_kernel(x_flat, jnp.asarray(_DST), table)
    return out.reshape(L, B, D).transpose(1, 0, 2)


# speedup vs baseline: 9.2808x; 1.0411x over previous
import functools
import math

import jax
import jax.numpy as jnp
from jax import lax
from jax.experimental import pallas as pl
from jax.experimental.pallas import tpu as pltpu
from jax.experimental.pallas import tpu_sc as plsc

VOCAB = 100000
D = 128
B = 4096
L = 50

NC = 2
NS = 16
LANES = 16
NW = NC * NS

BW = B // NW
CHUNKS = BW
NB = 8

SCALE = math.sqrt(D)

_mesh = plsc.VectorSubcoreMesh(core_axis_name="c", subcore_axis_name="s")

_scratch = (
    [pltpu.VMEM((BW, L), jnp.int32)]
    + [pltpu.VMEM((L, 1, D), jnp.float32) for _ in range(NB)]
    + [pltpu.SemaphoreType.DMA for _ in range(2 * NB)]
)


@functools.partial(
    pl.kernel,
    out_type=jax.ShapeDtypeStruct((L, B, D), jnp.float32),
    mesh=_mesh,
    scratch_types=_scratch,
    compiler_params=pltpu.CompilerParams(use_tc_tiling_on_sc=True),
)
def _emb_kernel(x_hbm, table_hbm, out_hbm, idx_v, *scratch):
    bufs = scratch[:NB]
    gsems = scratch[NB:2 * NB]
    osems = scratch[2 * NB:]

    wid = lax.axis_index("s") * NC + lax.axis_index("c")
    batch_base = wid * BW
    pltpu.sync_copy(x_hbm.at[pl.ds(batch_base, BW)], idx_v)

    def start_gather(c, s):
        pltpu.async_copy(table_hbm.at[idx_v.at[c]], bufs[s].at[:, 0], gsems[s])

    def wait_gather(c, s):
        pltpu.make_async_copy(
            table_hbm.at[idx_v.at[c]], bufs[s].at[:, 0], gsems[s]
        ).wait()

    def start_out(c, s):
        pltpu.async_copy(bufs[s], out_hbm.at[:, pl.ds(batch_base + c, 1)], osems[s])

    def wait_out(c, s):
        pltpu.make_async_copy(
            bufs[s], out_hbm.at[:, pl.ds(batch_base + c, 1)], osems[s]
        ).wait()

    def scale(s):
        buf = bufs[s]

        @plsc.parallel_loop(0, L, unroll=2)
        def _(i):
            for j in range(D // LANES):
                sl = pl.ds(j * LANES, LANES)
                buf[i, 0, sl] = buf[i, 0, sl] * SCALE

    for c in range(NB - 1):
        start_gather(c, c)
    wait_gather(0, 0)
    scale(0)
    start_out(0, 0)
    start_gather(NB - 1, NB - 1)

    def outer(g, carry):
        for b in range(NB):
            c = 1 + g * NB + b
            s = (1 + b) % NB
            wait_gather(c, s)
            scale(s)
            start_out(c, s)
            s2 = (s + NB - 1) % NB
            wait_out(c - 1, s2)
            start_gather(c + NB - 1, s2)
        return carry

    lax.fori_loop(0, (CHUNKS - NB) // NB, outer, 0)

    for c in range(CHUNKS - NB + 1, CHUNKS):
        s = c % NB
        wait_gather(c, s)
        scale(s)
        start_out(c, s)
    for c in range(CHUNKS - NB, CHUNKS):
        wait_out(c, c % NB)


def kernel(x, table):
    out = _emb_kernel(x.astype(jnp.int32), table)
    return out.transpose(1, 0, 2)
